# baseline (speedup 1.0000x reference)
import functools

import jax
import jax.numpy as jnp
from jax import lax
from jax.experimental import pallas as pl
from jax.experimental.pallas import tpu as pltpu
from jax.experimental.pallas import tpu_sc as plsc

D = 2048
FF = 4096
E = 8
K = 2
T = 2048
TK = T * K

TM = 512
TF = 512
NT = TK // TM + E - 1
R_PAD = NT * TM
NF = FF // TF

NW = 32
GCH = 16
CCH = 32


def _router_body(x_ref, wr_ref, out_ref):
    x = x_ref[...]
    wr = wr_ref[...]
    logits = lax.dot_general(x, wr, (((1,), (1,)), ((), ())),
                             preferred_element_type=jnp.float32)
    m1 = jnp.max(logits, axis=1, keepdims=True)
    i1 = jnp.argmax(logits, axis=1).reshape(T, 1)
    col = lax.broadcasted_iota(jnp.int32, (T, E), 1)
    masked = jnp.where(col == i1, -jnp.inf, logits)
    m2 = jnp.max(masked, axis=1, keepdims=True)
    i2 = jnp.argmax(masked, axis=1).reshape(T, 1)
    e21 = jnp.exp(m2 - m1)
    w1 = 1.0 / (1.0 + e21)
    w2 = 1.0 - w1
    out_ref[...] = jnp.concatenate(
        [w1, w2, i1.astype(jnp.float32), i2.astype(jnp.float32),
         jnp.zeros((T, 4), jnp.float32)], axis=1)


def _router(x_flat, wr):
    return pl.pallas_call(
        _router_body,
        out_shape=jax.ShapeDtypeStruct((T, E), jnp.float32),
    )(x_flat, wr)


_SC_MESH = plsc.VectorSubcoreMesh(core_axis_name="c", subcore_axis_name="s")


@functools.partial(
    pl.kernel,
    mesh=_SC_MESH,
    out_type=jax.ShapeDtypeStruct((R_PAD, D), jnp.float32),
    scratch_types=[
        pltpu.VMEM((2, GCH), jnp.int32),
        pltpu.VMEM((2, GCH, D), jnp.float32),
        pltpu.SemaphoreType.DMA((2,)),
        pltpu.SemaphoreType.DMA((2,)),
    ],
)
def _sc_gather(x_hbm, src_hbm, out_hbm, idx_v, rows_v, gsem, wsem):
    wid = lax.axis_index("s") * 2 + lax.axis_index("c")
    nc = R_PAD // NW // GCH
    base = wid * (R_PAD // NW)
    gathers = [None] * nc
    writes = [None] * nc
    for c in range(nc):
        b = c % 2
        if c >= 2:
            writes[c - 2].wait()
        pltpu.sync_copy(src_hbm.at[pl.ds(base + c * GCH, GCH)], idx_v.at[b])
        gathers[c] = pltpu.async_copy(x_hbm.at[idx_v.at[b]], rows_v.at[b],
                                      gsem.at[b])
        if c >= 1:
            gathers[c - 1].wait()
            writes[c - 1] = pltpu.async_copy(
                rows_v.at[(c - 1) % 2],
                out_hbm.at[pl.ds(base + (c - 1) * GCH, GCH)],
                wsem.at[(c - 1) % 2])
    gathers[nc - 1].wait()
    writes[nc - 1] = pltpu.async_copy(
        rows_v.at[(nc - 1) % 2],
        out_hbm.at[pl.ds(base + (nc - 1) * GCH, GCH)],
        wsem.at[(nc - 1) % 2])
    writes[nc - 2].wait()
    writes[nc - 1].wait()


@functools.partial(
    pl.kernel,
    mesh=_SC_MESH,
    out_type=(jax.ShapeDtypeStruct((T, D), jnp.float32),
              jax.ShapeDtypeStruct((T, D), jnp.float32)),
    scratch_types=[
        pltpu.VMEM((CCH,), jnp.int32),
        pltpu.VMEM((CCH, D), jnp.float32),
        pltpu.SemaphoreType.DMA,
    ],
)
def _sc_combine(h_hbm, p0_hbm, p1_hbm, a_hbm, b_hbm, idx_v, rows_v, sem):
    wid = lax.axis_index("s") * 2 + lax.axis_index("c")
    base = wid * (T // NW)
    for c in range(T // NW // CCH):
        o = base + c * CCH
        pltpu.sync_copy(p0_hbm.at[pl.ds(o, CCH)], idx_v)
        pltpu.async_copy(h_hbm.at[idx_v], rows_v, sem).wait()
        pltpu.sync_copy(rows_v, a_hbm.at[pl.ds(o, CCH)])
        pltpu.sync_copy(p1_hbm.at[pl.ds(o, CCH)], idx_v)
        pltpu.async_copy(h_hbm.at[idx_v], rows_v, sem).wait()
        pltpu.sync_copy(rows_v, b_hbm.at[pl.ds(o, CCH)])


def _ffn_body(te_ref, us_ref, xs_ref, w1_ref, w2_ref, out_ref):
    m = pl.program_id(0)
    ff = pl.program_id(1)

    @pl.when(us_ref[m] == 1)
    def _compute():
        x = xs_ref[...].astype(jnp.bfloat16)
        h = lax.dot_general(x, w1_ref[0],
                            (((1,), (0,)), ((), ())),
                            preferred_element_type=jnp.float32)
        h = 0.5 * h * (1.0 + lax.erf(h * 0.7071067811865476))
        prod = lax.dot_general(h.astype(jnp.bfloat16),
                               w2_ref[0],
                               (((1,), (0,)), ((), ())),
                               preferred_element_type=jnp.float32)

        @pl.when(ff == 0)
        def _first():
            out_ref[...] = prod

        @pl.when(ff > 0)
        def _rest():
            out_ref[...] += prod


def _grouped_ffn(tile_expert, used, xs, w1, w2):
    grid_spec = pltpu.PrefetchScalarGridSpec(
        num_scalar_prefetch=2,
        grid=(NT, NF),
        in_specs=[
            pl.BlockSpec((TM, D), lambda m, f, te, us: (m, 0)),
            pl.BlockSpec(
                (1, D, TF),
                lambda m, f, te, us: (te[m], 0,
                                      jnp.where(us[m] == 1, f, NF - 1))),
            pl.BlockSpec(
                (1, TF, D),
                lambda m, f, te, us: (te[m],
                                      jnp.where(us[m] == 1, f, NF - 1), 0)),
        ],
        out_specs=pl.BlockSpec((TM, D), lambda m, f, te, us: (m, 0)),
    )
    return pl.pallas_call(
        _ffn_body,
        grid_spec=grid_spec,
        out_shape=jax.ShapeDtypeStruct((R_PAD, D), jnp.float32),
    )(tile_expert, used, xs, w1, w2)


def _add_body(a_ref, b_ref, g0_ref, g1_ref, o_ref):
    o_ref[...] = (a_ref[...] * g0_ref[...][:, 0:1]
                  + b_ref[...] * g1_ref[...][:, 0:1])


def _combine_add(a, b, g0, g1):
    bs = lambda i: (i, 0)
    return pl.pallas_call(
        _add_body,
        grid=(8,),
        in_specs=[pl.BlockSpec((T // 8, D), bs),
                  pl.BlockSpec((T // 8, D), bs),
                  pl.BlockSpec((T // 8, 128), bs),
                  pl.BlockSpec((T // 8, 128), bs)],
        out_specs=pl.BlockSpec((T // 8, D), bs),
        out_shape=jax.ShapeDtypeStruct((T, D), jnp.float32),
    )(a, b, g0, g1)


def kernel(x, Wr, W1, W2):
    Bs, Ts, Dm = x.shape
    x_flat = x.reshape(Ts, Dm)

    r = _router(x_flat, Wr)
    w_flat = jnp.concatenate([r[:, 0], r[:, 1]])
    e_flat = jnp.concatenate([r[:, 2], r[:, 3]]).astype(jnp.int32)

    perm = jnp.argsort(e_flat, stable=True)
    e_sorted = e_flat[perm]
    tok_sorted = (perm % T).astype(jnp.int32)
    counts = jnp.bincount(e_flat, length=E)
    starts = jnp.concatenate(
        [jnp.zeros(1, counts.dtype), jnp.cumsum(counts)[:-1]])
    pcounts = ((counts + TM - 1) // TM) * TM
    pstarts = jnp.concatenate(
        [jnp.zeros(1, pcounts.dtype), jnp.cumsum(pcounts)[:-1]])
    rank = jnp.arange(TK) - starts[e_sorted]
    dst = (pstarts[e_sorted] + rank).astype(jnp.int32)
    src_full = (jnp.arange(R_PAD, dtype=jnp.int32) % T).at[dst].set(tok_sorted)
    pos = jnp.zeros(TK, jnp.int32).at[perm].set(dst)
    pos0, pos1 = pos[:T], pos[T:]
    pends = jnp.cumsum(pcounts)
    tile_expert = jnp.minimum(
        jnp.searchsorted(pends, jnp.arange(NT) * TM, side="right"),
        E - 1).astype(jnp.int32)
    used = (jnp.arange(NT) * TM < pends[-1]).astype(jnp.int32)

    xs = _sc_gather(x_flat, src_full)
    h = _grouped_ffn(tile_expert, used, xs,
                     W1.astype(jnp.bfloat16), W2.astype(jnp.bfloat16))
    a, b = _sc_combine(h, pos0, pos1)
    g0 = jnp.tile(r[:, 0:1], (1, 128))
    g1 = jnp.tile(r[:, 1:2], (1, 128))
    out = _combine_add(a, b, g0, g1)
    return out.reshape(Bs, Ts, Dm)


# speedup vs baseline: 1.4173x; 1.4173x over previous
import functools

import jax
import jax.numpy as jnp
from jax import lax
from jax.experimental import pallas as pl
from jax.experimental.pallas import tpu as pltpu
from jax.experimental.pallas import tpu_sc as plsc

D = 2048
FF = 4096
E = 8
K = 2
T = 2048
TK = T * K

TM = 512
TF = 512
NT = TK // TM + E - 1
R_PAD = NT * TM
NF = FF // TF

NW = 32
GCH = 16
CCH = 32


def _router_body(x_ref, wr_ref, out_ref):
    x = x_ref[...]
    wr = wr_ref[...]
    logits = lax.dot_general(x, wr, (((1,), (1,)), ((), ())),
                             preferred_element_type=jnp.float32)
    m1 = jnp.max(logits, axis=1, keepdims=True)
    i1 = jnp.argmax(logits, axis=1).reshape(T, 1)
    col = lax.broadcasted_iota(jnp.int32, (T, E), 1)
    masked = jnp.where(col == i1, -jnp.inf, logits)
    m2 = jnp.max(masked, axis=1, keepdims=True)
    i2 = jnp.argmax(masked, axis=1).reshape(T, 1)
    e21 = jnp.exp(m2 - m1)
    w1 = 1.0 / (1.0 + e21)
    w2 = 1.0 - w1
    out_ref[...] = jnp.concatenate(
        [w1, w2, i1.astype(jnp.float32), i2.astype(jnp.float32),
         jnp.zeros((T, 4), jnp.float32)], axis=1)


def _router(x_flat, wr):
    return pl.pallas_call(
        _router_body,
        out_shape=jax.ShapeDtypeStruct((T, E), jnp.float32),
    )(x_flat, wr)


_SC_MESH = plsc.VectorSubcoreMesh(core_axis_name="c", subcore_axis_name="s")


@functools.partial(
    pl.kernel,
    mesh=_SC_MESH,
    out_type=jax.ShapeDtypeStruct((R_PAD, D), jnp.float32),
    scratch_types=[
        pltpu.VMEM((2, GCH), jnp.int32),
        pltpu.VMEM((2, GCH, D), jnp.float32),
        pltpu.SemaphoreType.DMA((2,)),
        pltpu.SemaphoreType.DMA((2,)),
    ],
)
def _sc_gather(x_hbm, src_hbm, out_hbm, idx_v, rows_v, gsem, wsem):
    wid = lax.axis_index("s") * 2 + lax.axis_index("c")
    nc = R_PAD // NW // GCH
    base = wid * (R_PAD // NW)
    gathers = [None] * nc
    writes = [None] * nc
    for c in range(nc):
        b = c % 2
        if c >= 2:
            writes[c - 2].wait()
        pltpu.sync_copy(src_hbm.at[pl.ds(base + c * GCH, GCH)], idx_v.at[b])
        gathers[c] = pltpu.async_copy(x_hbm.at[idx_v.at[b]], rows_v.at[b],
                                      gsem.at[b])
        if c >= 1:
            gathers[c - 1].wait()
            writes[c - 1] = pltpu.async_copy(
                rows_v.at[(c - 1) % 2],
                out_hbm.at[pl.ds(base + (c - 1) * GCH, GCH)],
                wsem.at[(c - 1) % 2])
    gathers[nc - 1].wait()
    writes[nc - 1] = pltpu.async_copy(
        rows_v.at[(nc - 1) % 2],
        out_hbm.at[pl.ds(base + (nc - 1) * GCH, GCH)],
        wsem.at[(nc - 1) % 2])
    writes[nc - 2].wait()
    writes[nc - 1].wait()


@functools.partial(
    pl.kernel,
    mesh=_SC_MESH,
    out_type=(jax.ShapeDtypeStruct((T, D), jnp.float32),
              jax.ShapeDtypeStruct((T, D), jnp.float32)),
    scratch_types=[
        pltpu.VMEM((CCH,), jnp.int32),
        pltpu.VMEM((CCH, D), jnp.float32),
        pltpu.SemaphoreType.DMA,
    ],
)
def _sc_combine(h_hbm, p0_hbm, p1_hbm, a_hbm, b_hbm, idx_v, rows_v, sem):
    wid = lax.axis_index("s") * 2 + lax.axis_index("c")
    base = wid * (T // NW)
    for c in range(T // NW // CCH):
        o = base + c * CCH
        pltpu.sync_copy(p0_hbm.at[pl.ds(o, CCH)], idx_v)
        pltpu.async_copy(h_hbm.at[idx_v], rows_v, sem).wait()
        pltpu.sync_copy(rows_v, a_hbm.at[pl.ds(o, CCH)])
        pltpu.sync_copy(p1_hbm.at[pl.ds(o, CCH)], idx_v)
        pltpu.async_copy(h_hbm.at[idx_v], rows_v, sem).wait()
        pltpu.sync_copy(rows_v, b_hbm.at[pl.ds(o, CCH)])


def _ffn_body(te_ref, us_ref, xs_ref, w1_ref, w2_ref, out_ref):
    m = pl.program_id(0)
    ff = pl.program_id(1)

    @pl.when(us_ref[m] == 1)
    def _compute():
        x = xs_ref[...].astype(jnp.bfloat16)
        h = lax.dot_general(x, w1_ref[0].astype(jnp.bfloat16),
                            (((1,), (0,)), ((), ())),
                            preferred_element_type=jnp.float32)
        h = 0.5 * h * (1.0 + lax.erf(h * 0.7071067811865476))
        prod = lax.dot_general(h.astype(jnp.bfloat16),
                               w2_ref[0].astype(jnp.bfloat16),
                               (((1,), (0,)), ((), ())),
                               preferred_element_type=jnp.float32)

        @pl.when(ff == 0)
        def _first():
            out_ref[...] = prod

        @pl.when(ff > 0)
        def _rest():
            out_ref[...] += prod


def _grouped_ffn(tile_expert, used, xs, w1, w2):
    grid_spec = pltpu.PrefetchScalarGridSpec(
        num_scalar_prefetch=2,
        grid=(NT, NF),
        in_specs=[
            pl.BlockSpec((TM, D), lambda m, f, te, us: (m, 0)),
            pl.BlockSpec(
                (1, D, TF),
                lambda m, f, te, us: (te[m], 0,
                                      jnp.where(us[m] == 1, f, NF - 1))),
            pl.BlockSpec(
                (1, TF, D),
                lambda m, f, te, us: (te[m],
                                      jnp.where(us[m] == 1, f, NF - 1), 0)),
        ],
        out_specs=pl.BlockSpec((TM, D), lambda m, f, te, us: (m, 0)),
    )
    return pl.pallas_call(
        _ffn_body,
        grid_spec=grid_spec,
        out_shape=jax.ShapeDtypeStruct((R_PAD, D), jnp.float32),
    )(tile_expert, used, xs, w1, w2)


def _add_body(a_ref, b_ref, g0_ref, g1_ref, o_ref):
    o_ref[...] = (a_ref[...] * g0_ref[...][:, 0:1]
                  + b_ref[...] * g1_ref[...][:, 0:1])


def _combine_add(a, b, g0, g1):
    bs = lambda i: (i, 0)
    return pl.pallas_call(
        _add_body,
        grid=(8,),
        in_specs=[pl.BlockSpec((T // 8, D), bs),
                  pl.BlockSpec((T // 8, D), bs),
                  pl.BlockSpec((T // 8, 128), bs),
                  pl.BlockSpec((T // 8, 128), bs)],
        out_specs=pl.BlockSpec((T // 8, D), bs),
        out_shape=jax.ShapeDtypeStruct((T, D), jnp.float32),
    )(a, b, g0, g1)


def kernel(x, Wr, W1, W2):
    Bs, Ts, Dm = x.shape
    x_flat = x.reshape(Ts, Dm)

    r = _router(x_flat, Wr)
    w_flat = jnp.concatenate([r[:, 0], r[:, 1]])
    e_flat = jnp.concatenate([r[:, 2], r[:, 3]]).astype(jnp.int32)

    perm = jnp.argsort(e_flat, stable=True)
    e_sorted = e_flat[perm]
    tok_sorted = (perm % T).astype(jnp.int32)
    counts = jnp.bincount(e_flat, length=E)
    starts = jnp.concatenate(
        [jnp.zeros(1, counts.dtype), jnp.cumsum(counts)[:-1]])
    pcounts = ((counts + TM - 1) // TM) * TM
    pstarts = jnp.concatenate(
        [jnp.zeros(1, pcounts.dtype), jnp.cumsum(pcounts)[:-1]])
    rank = jnp.arange(TK) - starts[e_sorted]
    dst = (pstarts[e_sorted] + rank).astype(jnp.int32)
    src_full = (jnp.arange(R_PAD, dtype=jnp.int32) % T).at[dst].set(tok_sorted)
    pos = jnp.zeros(TK, jnp.int32).at[perm].set(dst)
    pos0, pos1 = pos[:T], pos[T:]
    pends = jnp.cumsum(pcounts)
    tile_expert = jnp.minimum(
        jnp.searchsorted(pends, jnp.arange(NT) * TM, side="right"),
        E - 1).astype(jnp.int32)
    used = (jnp.arange(NT) * TM < pends[-1]).astype(jnp.int32)

    xs = _sc_gather(x_flat, src_full)
    h = _grouped_ffn(tile_expert, used, xs, W1, W2)
    a, b = _sc_combine(h, pos0, pos1)
    g0 = jnp.tile(r[:, 0:1], (1, 128))
    g1 = jnp.tile(r[:, 1:2], (1, 128))
    out = _combine_add(a, b, g0, g1)
    return out.reshape(Bs, Ts, Dm)


# speedup vs baseline: 1.6956x; 1.1964x over previous
import functools

import jax
import jax.numpy as jnp
from jax import lax
from jax.experimental import pallas as pl
from jax.experimental.pallas import tpu as pltpu
from jax.experimental.pallas import tpu_sc as plsc

D = 2048
FF = 4096
E = 8
K = 2
T = 2048
TK = T * K

TM = 640
TF = 512
NT = TK // TM + E - 1
R_PAD = NT * TM
GPAD = 8448
NF = FF // TF

NW = 32
GCH = 24
CCH = 32


def _router_body(x_ref, wr_ref, out_ref):
    x = x_ref[...]
    wr = wr_ref[...]
    logits = lax.dot_general(x, wr, (((1,), (1,)), ((), ())),
                             preferred_element_type=jnp.float32)
    m1 = jnp.max(logits, axis=1, keepdims=True)
    i1 = jnp.argmax(logits, axis=1).reshape(T, 1)
    col = lax.broadcasted_iota(jnp.int32, (T, E), 1)
    masked = jnp.where(col == i1, -jnp.inf, logits)
    m2 = jnp.max(masked, axis=1, keepdims=True)
    i2 = jnp.argmax(masked, axis=1).reshape(T, 1)
    e21 = jnp.exp(m2 - m1)
    w1 = 1.0 / (1.0 + e21)
    w2 = 1.0 - w1
    out_ref[...] = jnp.concatenate(
        [w1, w2, i1.astype(jnp.float32), i2.astype(jnp.float32),
         jnp.zeros((T, 4), jnp.float32)], axis=1)


def _router(x_flat, wr):
    return pl.pallas_call(
        _router_body,
        out_shape=jax.ShapeDtypeStruct((T, E), jnp.float32),
    )(x_flat, wr)


_SC_MESH = plsc.VectorSubcoreMesh(core_axis_name="c", subcore_axis_name="s")


@functools.partial(
    pl.kernel,
    mesh=_SC_MESH,
    out_type=jax.ShapeDtypeStruct((GPAD, D), jnp.float32),
    scratch_types=[
        pltpu.VMEM((2, GCH), jnp.int32),
        pltpu.VMEM((2, GCH, D), jnp.float32),
        pltpu.SemaphoreType.DMA((2,)),
        pltpu.SemaphoreType.DMA((2,)),
    ],
)
def _sc_gather(x_hbm, src_hbm, out_hbm, idx_v, rows_v, gsem, wsem):
    wid = lax.axis_index("s") * 2 + lax.axis_index("c")
    nc = GPAD // NW // GCH
    base = wid * (GPAD // NW)
    gathers = [None] * nc
    writes = [None] * nc
    for c in range(nc):
        b = c % 2
        if c >= 2:
            writes[c - 2].wait()
        pltpu.sync_copy(src_hbm.at[pl.ds(base + c * GCH, GCH)], idx_v.at[b])
        gathers[c] = pltpu.async_copy(x_hbm.at[idx_v.at[b]], rows_v.at[b],
                                      gsem.at[b])
        if c >= 1:
            gathers[c - 1].wait()
            writes[c - 1] = pltpu.async_copy(
                rows_v.at[(c - 1) % 2],
                out_hbm.at[pl.ds(base + (c - 1) * GCH, GCH)],
                wsem.at[(c - 1) % 2])
    gathers[nc - 1].wait()
    writes[nc - 1] = pltpu.async_copy(
        rows_v.at[(nc - 1) % 2],
        out_hbm.at[pl.ds(base + (nc - 1) * GCH, GCH)],
        wsem.at[(nc - 1) % 2])
    writes[nc - 2].wait()
    writes[nc - 1].wait()


@functools.partial(
    pl.kernel,
    mesh=_SC_MESH,
    out_type=(jax.ShapeDtypeStruct((T, D), jnp.float32),
              jax.ShapeDtypeStruct((T, D), jnp.float32)),
    scratch_types=[
        pltpu.VMEM((CCH,), jnp.int32),
        pltpu.VMEM((CCH, D), jnp.float32),
        pltpu.SemaphoreType.DMA,
    ],
)
def _sc_combine(h_hbm, p0_hbm, p1_hbm, a_hbm, b_hbm, idx_v, rows_v, sem):
    wid = lax.axis_index("s") * 2 + lax.axis_index("c")
    base = wid * (T // NW)
    for c in range(T // NW // CCH):
        o = base + c * CCH
        pltpu.sync_copy(p0_hbm.at[pl.ds(o, CCH)], idx_v)
        pltpu.async_copy(h_hbm.at[idx_v], rows_v, sem).wait()
        pltpu.sync_copy(rows_v, a_hbm.at[pl.ds(o, CCH)])
        pltpu.sync_copy(p1_hbm.at[pl.ds(o, CCH)], idx_v)
        pltpu.async_copy(h_hbm.at[idx_v], rows_v, sem).wait()
        pltpu.sync_copy(rows_v, b_hbm.at[pl.ds(o, CCH)])


def _ffn_body(te_ref, us_ref, xs_ref, w1_ref, w2_ref, out_ref):
    m = pl.program_id(0)
    ff = pl.program_id(1)

    @pl.when(us_ref[m] == 1)
    def _compute():
        x = xs_ref[...].astype(jnp.bfloat16)
        h = lax.dot_general(x, w1_ref[0].astype(jnp.bfloat16),
                            (((1,), (0,)), ((), ())),
                            preferred_element_type=jnp.float32)
        h = 0.5 * h * (1.0 + lax.erf(h * 0.7071067811865476))
        prod = lax.dot_general(h.astype(jnp.bfloat16),
                               w2_ref[0].astype(jnp.bfloat16),
                               (((1,), (0,)), ((), ())),
                               preferred_element_type=jnp.float32)

        @pl.when(ff == 0)
        def _first():
            out_ref[...] = prod

        @pl.when(ff > 0)
        def _rest():
            out_ref[...] += prod


def _grouped_ffn(tile_expert, used, xs, w1, w2):
    grid_spec = pltpu.PrefetchScalarGridSpec(
        num_scalar_prefetch=2,
        grid=(NT, NF),
        in_specs=[
            pl.BlockSpec((TM, D), lambda m, f, te, us: (m, 0)),
            pl.BlockSpec(
                (1, D, TF),
                lambda m, f, te, us: (te[m], 0,
                                      jnp.where(us[m] == 1, f, NF - 1))),
            pl.BlockSpec(
                (1, TF, D),
                lambda m, f, te, us: (te[m],
                                      jnp.where(us[m] == 1, f, NF - 1), 0)),
        ],
        out_specs=pl.BlockSpec((TM, D), lambda m, f, te, us: (m, 0)),
    )
    return pl.pallas_call(
        _ffn_body,
        grid_spec=grid_spec,
        out_shape=jax.ShapeDtypeStruct((R_PAD, D), jnp.float32),
    )(tile_expert, used, xs, w1, w2)


def _add_body(a_ref, b_ref, g0_ref, g1_ref, o_ref):
    o_ref[...] = (a_ref[...] * g0_ref[...][:, 0:1]
                  + b_ref[...] * g1_ref[...][:, 0:1])


def _combine_add(a, b, g0, g1):
    bs = lambda i: (i, 0)
    return pl.pallas_call(
        _add_body,
        grid=(8,),
        in_specs=[pl.BlockSpec((T // 8, D), bs),
                  pl.BlockSpec((T // 8, D), bs),
                  pl.BlockSpec((T // 8, 128), bs),
                  pl.BlockSpec((T // 8, 128), bs)],
        out_specs=pl.BlockSpec((T // 8, D), bs),
        out_shape=jax.ShapeDtypeStruct((T, D), jnp.float32),
    )(a, b, g0, g1)


def kernel(x, Wr, W1, W2):
    Bs, Ts, Dm = x.shape
    x_flat = x.reshape(Ts, Dm)

    r = _router(x_flat, Wr)
    w_flat = jnp.concatenate([r[:, 0], r[:, 1]])
    e_flat = jnp.concatenate([r[:, 2], r[:, 3]]).astype(jnp.int32)

    perm = jnp.argsort(e_flat, stable=True)
    e_sorted = e_flat[perm]
    tok_sorted = (perm % T).astype(jnp.int32)
    counts = jnp.bincount(e_flat, length=E)
    starts = jnp.concatenate(
        [jnp.zeros(1, counts.dtype), jnp.cumsum(counts)[:-1]])
    pcounts = ((counts + TM - 1) // TM) * TM
    pstarts = jnp.concatenate(
        [jnp.zeros(1, pcounts.dtype), jnp.cumsum(pcounts)[:-1]])
    rank = jnp.arange(TK) - starts[e_sorted]
    dst = (pstarts[e_sorted] + rank).astype(jnp.int32)
    src_full = (jnp.arange(GPAD, dtype=jnp.int32) % T).at[dst].set(tok_sorted)
    pos = jnp.zeros(TK, jnp.int32).at[perm].set(dst)
    pos0, pos1 = pos[:T], pos[T:]
    pends = jnp.cumsum(pcounts)
    tile_expert = jnp.minimum(
        jnp.searchsorted(pends, jnp.arange(NT) * TM, side="right"),
        E - 1).astype(jnp.int32)
    used = (jnp.arange(NT) * TM < pends[-1]).astype(jnp.int32)

    xs = _sc_gather(x_flat, src_full)
    h = _grouped_ffn(tile_expert, used, xs, W1, W2)
    a, b = _sc_combine(h, pos0, pos1)
    g0 = jnp.tile(r[:, 0:1], (1, 128))
    g1 = jnp.tile(r[:, 1:2], (1, 128))
    out = _combine_add(a, b, g0, g1)
    return out.reshape(Bs, Ts, Dm)


# speedup vs baseline: 1.7536x; 1.0342x over previous
import functools

import jax
import jax.numpy as jnp
from jax import lax
from jax.experimental import pallas as pl
from jax.experimental.pallas import tpu as pltpu
from jax.experimental.pallas import tpu_sc as plsc

D = 2048
FF = 4096
E = 8
K = 2
T = 2048
TK = T * K

TM = 576
TF = 512
NT = TK // TM + E - 1
R_PAD = NT * TM
GPAD = 8192
NF = FF // TF

NW = 32
GCH = 16
CCH = 32


def _router_body(x_ref, wr_ref, out_ref):
    x = x_ref[...]
    wr = wr_ref[...]
    logits = lax.dot_general(x, wr, (((1,), (1,)), ((), ())),
                             preferred_element_type=jnp.float32)
    m1 = jnp.max(logits, axis=1, keepdims=True)
    i1 = jnp.argmax(logits, axis=1).reshape(T, 1)
    col = lax.broadcasted_iota(jnp.int32, (T, E), 1)
    masked = jnp.where(col == i1, -jnp.inf, logits)
    m2 = jnp.max(masked, axis=1, keepdims=True)
    i2 = jnp.argmax(masked, axis=1).reshape(T, 1)
    e21 = jnp.exp(m2 - m1)
    w1 = 1.0 / (1.0 + e21)
    w2 = 1.0 - w1
    out_ref[...] = jnp.concatenate(
        [w1, w2, i1.astype(jnp.float32), i2.astype(jnp.float32),
         jnp.zeros((T, 4), jnp.float32)], axis=1)


def _router(x_flat, wr):
    return pl.pallas_call(
        _router_body,
        out_shape=jax.ShapeDtypeStruct((T, E), jnp.float32),
    )(x_flat, wr)


_SC_MESH = plsc.VectorSubcoreMesh(core_axis_name="c", subcore_axis_name="s")


@functools.partial(
    pl.kernel,
    mesh=_SC_MESH,
    out_type=jax.ShapeDtypeStruct((GPAD, D), jnp.float32),
    scratch_types=[
        pltpu.VMEM((2, GCH), jnp.int32),
        pltpu.VMEM((2, GCH, D), jnp.float32),
        pltpu.SemaphoreType.DMA((2,)),
        pltpu.SemaphoreType.DMA((2,)),
    ],
)
def _sc_gather(x_hbm, src_hbm, out_hbm, idx_v, rows_v, gsem, wsem):
    wid = lax.axis_index("s") * 2 + lax.axis_index("c")
    nc = GPAD // NW // GCH
    base = wid * (GPAD // NW)
    gathers = [None] * nc
    writes = [None] * nc
    for c in range(nc):
        b = c % 2
        if c >= 2:
            writes[c - 2].wait()
        pltpu.sync_copy(src_hbm.at[pl.ds(base + c * GCH, GCH)], idx_v.at[b])
        gathers[c] = pltpu.async_copy(x_hbm.at[idx_v.at[b]], rows_v.at[b],
                                      gsem.at[b])
        if c >= 1:
            gathers[c - 1].wait()
            writes[c - 1] = pltpu.async_copy(
                rows_v.at[(c - 1) % 2],
                out_hbm.at[pl.ds(base + (c - 1) * GCH, GCH)],
                wsem.at[(c - 1) % 2])
    gathers[nc - 1].wait()
    writes[nc - 1] = pltpu.async_copy(
        rows_v.at[(nc - 1) % 2],
        out_hbm.at[pl.ds(base + (nc - 1) * GCH, GCH)],
        wsem.at[(nc - 1) % 2])
    writes[nc - 2].wait()
    writes[nc - 1].wait()


@functools.partial(
    pl.kernel,
    mesh=_SC_MESH,
    out_type=(jax.ShapeDtypeStruct((T, D), jnp.float32),
              jax.ShapeDtypeStruct((T, D), jnp.float32)),
    scratch_types=[
        pltpu.VMEM((CCH,), jnp.int32),
        pltpu.VMEM((CCH, D), jnp.float32),
        pltpu.SemaphoreType.DMA,
    ],
)
def _sc_combine(h_hbm, p0_hbm, p1_hbm, a_hbm, b_hbm, idx_v, rows_v, sem):
    wid = lax.axis_index("s") * 2 + lax.axis_index("c")
    base = wid * (T // NW)
    for c in range(T // NW // CCH):
        o = base + c * CCH
        pltpu.sync_copy(p0_hbm.at[pl.ds(o, CCH)], idx_v)
        pltpu.async_copy(h_hbm.at[idx_v], rows_v, sem).wait()
        pltpu.sync_copy(rows_v, a_hbm.at[pl.ds(o, CCH)])
        pltpu.sync_copy(p1_hbm.at[pl.ds(o, CCH)], idx_v)
        pltpu.async_copy(h_hbm.at[idx_v], rows_v, sem).wait()
        pltpu.sync_copy(rows_v, b_hbm.at[pl.ds(o, CCH)])


def _ffn_body(te_ref, us_ref, xs_ref, w1_ref, w2_ref, out_ref):
    m = pl.program_id(0)
    ff = pl.program_id(1)

    @pl.when(us_ref[m] == 1)
    def _compute():
        x = xs_ref[...].astype(jnp.bfloat16)
        h = lax.dot_general(x, w1_ref[0].astype(jnp.bfloat16),
                            (((1,), (0,)), ((), ())),
                            preferred_element_type=jnp.float32)
        h = 0.5 * h * (1.0 + lax.erf(h * 0.7071067811865476))
        prod = lax.dot_general(h.astype(jnp.bfloat16),
                               w2_ref[0].astype(jnp.bfloat16),
                               (((1,), (0,)), ((), ())),
                               preferred_element_type=jnp.float32)

        @pl.when(ff == 0)
        def _first():
            out_ref[...] = prod

        @pl.when(ff > 0)
        def _rest():
            out_ref[...] += prod


def _grouped_ffn(tile_expert, used, xs, w1, w2):
    grid_spec = pltpu.PrefetchScalarGridSpec(
        num_scalar_prefetch=2,
        grid=(NT, NF),
        in_specs=[
            pl.BlockSpec((TM, D), lambda m, f, te, us: (m, 0)),
            pl.BlockSpec(
                (1, D, TF),
                lambda m, f, te, us: (te[m], 0,
                                      jnp.where(us[m] == 1, f, NF - 1))),
            pl.BlockSpec(
                (1, TF, D),
                lambda m, f, te, us: (te[m],
                                      jnp.where(us[m] == 1, f, NF - 1), 0)),
        ],
        out_specs=pl.BlockSpec((TM, D), lambda m, f, te, us: (m, 0)),
    )
    return pl.pallas_call(
        _ffn_body,
        grid_spec=grid_spec,
        out_shape=jax.ShapeDtypeStruct((R_PAD, D), jnp.float32),
    )(tile_expert, used, xs, w1, w2)


def _add_body(a_ref, b_ref, g0_ref, g1_ref, o_ref):
    o_ref[...] = (a_ref[...] * g0_ref[...][:, 0:1]
                  + b_ref[...] * g1_ref[...][:, 0:1])


def _combine_add(a, b, g0, g1):
    bs = lambda i: (i, 0)
    return pl.pallas_call(
        _add_body,
        grid=(8,),
        in_specs=[pl.BlockSpec((T // 8, D), bs),
                  pl.BlockSpec((T // 8, D), bs),
                  pl.BlockSpec((T // 8, 128), bs),
                  pl.BlockSpec((T // 8, 128), bs)],
        out_specs=pl.BlockSpec((T // 8, D), bs),
        out_shape=jax.ShapeDtypeStruct((T, D), jnp.float32),
    )(a, b, g0, g1)


def kernel(x, Wr, W1, W2):
    Bs, Ts, Dm = x.shape
    x_flat = x.reshape(Ts, Dm)

    r = _router(x_flat, Wr)
    w_flat = jnp.concatenate([r[:, 0], r[:, 1]])
    e_flat = jnp.concatenate([r[:, 2], r[:, 3]]).astype(jnp.int32)

    perm = jnp.argsort(e_flat, stable=True)
    e_sorted = e_flat[perm]
    tok_sorted = (perm % T).astype(jnp.int32)
    counts = jnp.bincount(e_flat, length=E)
    starts = jnp.concatenate(
        [jnp.zeros(1, counts.dtype), jnp.cumsum(counts)[:-1]])
    pcounts = ((counts + TM - 1) // TM) * TM
    pstarts = jnp.concatenate(
        [jnp.zeros(1, pcounts.dtype), jnp.cumsum(pcounts)[:-1]])
    rank = jnp.arange(TK) - starts[e_sorted]
    dst = (pstarts[e_sorted] + rank).astype(jnp.int32)
    src_full = (jnp.arange(GPAD, dtype=jnp.int32) % T).at[dst].set(tok_sorted)
    pos = jnp.zeros(TK, jnp.int32).at[perm].set(dst)
    pos0, pos1 = pos[:T], pos[T:]
    pends = jnp.cumsum(pcounts)
    tile_expert = jnp.minimum(
        jnp.searchsorted(pends, jnp.arange(NT) * TM, side="right"),
        E - 1).astype(jnp.int32)
    used = (jnp.arange(NT) * TM < pends[-1]).astype(jnp.int32)

    xs = _sc_gather(x_flat, src_full)
    h = _grouped_ffn(tile_expert, used, xs, W1, W2)
    a, b = _sc_combine(h, pos0, pos1)
    g0 = jnp.tile(r[:, 0:1], (1, 128))
    g1 = jnp.tile(r[:, 1:2], (1, 128))
    out = _combine_add(a, b, g0, g1)
    return out.reshape(Bs, Ts, Dm)
